# baseline (speedup 1.0000x reference)
import functools

import jax
import jax.numpy as jnp
from jax import lax
from jax.experimental import pallas as pl
from jax.experimental.pallas import tpu as pltpu
from jax.experimental.pallas import tpu_sc as plsc

_NUM_CORES = 2
_NUM_SUBCORES = 16
_NW = _NUM_CORES * _NUM_SUBCORES
_TB = 128
_NBUF = 2

_N_I1 = 200
_N_I0 = 4096
_D = 64
_HALF = _N_I1 // 10


def _gather_body(table_hbm, tt_hbm, out_hbm, *scratch, n_i1):
    idx_b = scratch[0:_NBUF]
    p_b = scratch[_NBUF:2 * _NBUF]
    rows_b = scratch[2 * _NBUF:3 * _NBUF]
    sem_i = scratch[3 * _NBUF:4 * _NBUF]
    sem_g = scratch[4 * _NBUF:5 * _NBUF]
    sem_w = scratch[5 * _NBUF:6 * _NBUF]

    w = lax.axis_index("s") * _NUM_CORES + lax.axis_index("c")

    def start_idx(k, s):
        pltpu.async_copy(tt_hbm.at[pl.ds(k * _N_I0 + _TB * w, _TB)],
                         idx_b[s], sem_i[s])

    def wait_idx(k, s):
        pltpu.make_async_copy(tt_hbm.at[pl.ds(k * _N_I0 + _TB * w, _TB)],
                              idx_b[s], sem_i[s]).wait()

    def stage_slots(s):
        @pl.loop(0, _TB // 16)
        def _jg(jg):
            v = idx_b[s][pl.ds(jg * 16, 16)]
            slot = (lax.shift_left(lax.shift_right_logical(v, 13), 13)
                    + lax.shift_left(lax.bitwise_and(v, 4095), 1)
                    + lax.bitwise_and(lax.shift_right_logical(v, 12), 1))
            p_b[s][pl.ds(jg * 16, 16)] = slot

    def start_gather(k, s):
        pltpu.async_copy(table_hbm.at[p_b[s]], rows_b[s], sem_g[s])

    def wait_gather(k, s):
        pltpu.make_async_copy(table_hbm.at[p_b[s]], rows_b[s], sem_g[s]).wait()

    def start_write(k, s):
        pltpu.async_copy(
            rows_b[s], out_hbm.at[k, pl.ds(w * _TB, _TB), pl.ds(0, _D)],
            sem_w[s])

    def wait_write(k, s):
        pltpu.make_async_copy(
            rows_b[s], out_hbm.at[k, pl.ds(w * _TB, _TB), pl.ds(0, _D)],
            sem_w[s]).wait()

    def unit(k, s, with_wait_w=True, load_ahead=True):
        wait_gather(k, s)
        if with_wait_w:
            wait_write(k - _NBUF, s)
        start_write(k, s)
        if load_ahead:
            start_idx(k + _NBUF, s)

    for k in range(_NBUF):
        start_idx(k, k)
    for k in range(_NBUF):
        wait_idx(k, k)
        stage_slots(k)
        start_gather(k, k)
    for k in range(_NBUF):
        unit(k, k, with_wait_w=False)
        wait_idx(k + _NBUF, k)
        stage_slots(k)
        start_gather(k + _NBUF, k)

    n_steady_groups = (n_i1 - 2 * _NBUF) // _NBUF

    @pl.loop(0, n_steady_groups)
    def _grp(g):
        for s in range(_NBUF):
            k = _NBUF + g * _NBUF + s
            unit(k, s)
            wait_idx(k + _NBUF, s)
            stage_slots(s)
            start_gather(k + _NBUF, s)

    for k in range(n_i1 - _NBUF, n_i1):
        unit(k, k % _NBUF, load_ahead=False)
    for k in range(n_i1 - _NBUF, n_i1):
        wait_write(k, k % _NBUF)


def _widen_body(et_ref, o_ref):
    xt = jnp.transpose(et_ref[...], (1, 0))
    c2 = xt.shape[0] // 2
    o_ref[:, :_D] = xt[:c2, :]
    o_ref[:, _D:] = xt[c2:, :]


def _plane_body(g_ref, o_ref):
    x = g_ref[...]
    xt = jnp.transpose(x, (0, 2, 1))
    o_ref[...] = xt[:, :_D, :]


def _acc_plane_body(g_ref, acc_ref, o_ref):
    xt = jnp.transpose(g_ref[...], (0, 2, 1))
    o_ref[...] = xt[:, :_D, :]


def kernel(token_ids, embeddings):
    nv, d = embeddings.shape
    n0, n1 = token_ids.shape
    tt = token_ids.T.reshape(-1).astype(jnp.int32)

    c = 8192
    table_p = pl.pallas_call(
        _widen_body,
        grid=(pl.cdiv(nv, c),),
        in_specs=[pl.BlockSpec((d, c), lambda i: (0, i))],
        out_specs=pl.BlockSpec((c // 2, 128), lambda i: (i, 0)),
        out_shape=jax.ShapeDtypeStruct(
            (pl.cdiv(nv, c) * (c // 2), 128), jnp.float32),
    )(embeddings.T)
    table_c = table_p.reshape(pl.cdiv(nv, c) * c, d)

    mesh = plsc.VectorSubcoreMesh(core_axis_name="c", subcore_axis_name="s")
    gather_rows = pl.kernel(
        functools.partial(_gather_body, n_i1=_HALF),
        out_type=jax.ShapeDtypeStruct((_HALF, n0, 128), jnp.float32),
        mesh=mesh,
        scratch_types=(
            [pltpu.VMEM((_TB,), jnp.int32) for _ in range(_NBUF)]
            + [pltpu.VMEM((_TB,), jnp.int32) for _ in range(_NBUF)]
            + [pltpu.VMEM((_TB, _D), jnp.float32) for _ in range(_NBUF)]
            + [pltpu.SemaphoreType.DMA for _ in range(3 * _NBUF)]
        ),
        compiler_params=pltpu.CompilerParams(
            use_tc_tiling_on_sc=False, needs_layout_passes=False),
    )
    piece_elems = _HALF * n0
    gs = [gather_rows(table_c, tt[j * piece_elems:(j + 1) * piece_elems])
          for j in range(_N_I1 // _HALF)]

    out_t = pl.pallas_call(
        _plane_body,
        grid=(_HALF,),
        in_specs=[pl.BlockSpec((1, n0, 128), lambda i: (i, 0, 0))],
        out_specs=pl.BlockSpec((1, _D, n0), lambda i: (i, 0, 0)),
        out_shape=jax.ShapeDtypeStruct((n1, _D, n0), jnp.float32),
    )(gs[0])
    for j in range(1, _N_I1 // _HALF):
        out_t = pl.pallas_call(
            _acc_plane_body,
            grid=(_HALF,),
            in_specs=[
                pl.BlockSpec((1, n0, 128), lambda i: (i, 0, 0)),
                pl.BlockSpec(memory_space=pltpu.HBM),
            ],
            out_specs=pl.BlockSpec(
                (1, _D, n0), lambda i, o=j * _HALF: (i + o, 0, 0)),
            out_shape=jax.ShapeDtypeStruct((n1, _D, n0), jnp.float32),
            input_output_aliases={1: 0},
        )(gs[j], out_t)
    return out_t.transpose(2, 0, 1)


# speedup vs baseline: 1.0504x; 1.0504x over previous
import functools

import jax
import jax.numpy as jnp
from jax import lax
from jax.experimental import pallas as pl
from jax.experimental.pallas import tpu as pltpu
from jax.experimental.pallas import tpu_sc as plsc

_NUM_CORES = 2
_NUM_SUBCORES = 16
_NW = _NUM_CORES * _NUM_SUBCORES
_TB = 128
_NBUF = 2

_N_I1 = 200
_C = 16384
_CLOG = 14
_N_I0 = 4096
_D = 64
_HALF = _N_I1 // 10


def _gather_body(table_hbm, tt_hbm, out_hbm, *scratch, n_i1):
    idx_b = scratch[0:_NBUF]
    p_b = scratch[_NBUF:2 * _NBUF]
    rows_b = scratch[2 * _NBUF:3 * _NBUF]
    sem_i = scratch[3 * _NBUF:4 * _NBUF]
    sem_g = scratch[4 * _NBUF:5 * _NBUF]
    sem_w = scratch[5 * _NBUF:6 * _NBUF]

    w = lax.axis_index("s") * _NUM_CORES + lax.axis_index("c")

    def start_idx(k, s):
        pltpu.async_copy(tt_hbm.at[pl.ds(k * _N_I0 + _TB * w, _TB)],
                         idx_b[s], sem_i[s])

    def wait_idx(k, s):
        pltpu.make_async_copy(tt_hbm.at[pl.ds(k * _N_I0 + _TB * w, _TB)],
                              idx_b[s], sem_i[s]).wait()

    def stage_slots(s):
        @pl.loop(0, _TB // 16)
        def _jg(jg):
            v = idx_b[s][pl.ds(jg * 16, 16)]
            slot = (lax.shift_left(lax.shift_right_logical(v, _CLOG), _CLOG)
                    + lax.shift_left(lax.bitwise_and(v, _C // 2 - 1), 1)
                    + lax.bitwise_and(lax.shift_right_logical(v, _CLOG - 1), 1))
            p_b[s][pl.ds(jg * 16, 16)] = slot

    def start_gather(k, s):
        pltpu.async_copy(table_hbm.at[p_b[s]], rows_b[s], sem_g[s])

    def wait_gather(k, s):
        pltpu.make_async_copy(table_hbm.at[p_b[s]], rows_b[s], sem_g[s]).wait()

    def start_write(k, s):
        pltpu.async_copy(
            rows_b[s], out_hbm.at[k, pl.ds(w * _TB, _TB), pl.ds(0, _D)],
            sem_w[s])

    def wait_write(k, s):
        pltpu.make_async_copy(
            rows_b[s], out_hbm.at[k, pl.ds(w * _TB, _TB), pl.ds(0, _D)],
            sem_w[s]).wait()

    def unit(k, s, with_wait_w=True, load_ahead=True):
        wait_gather(k, s)
        if with_wait_w:
            wait_write(k - _NBUF, s)
        start_write(k, s)
        if load_ahead:
            start_idx(k + _NBUF, s)

    for k in range(_NBUF):
        start_idx(k, k)
    for k in range(_NBUF):
        wait_idx(k, k)
        stage_slots(k)
        start_gather(k, k)
    for k in range(_NBUF):
        unit(k, k, with_wait_w=False)
        wait_idx(k + _NBUF, k)
        stage_slots(k)
        start_gather(k + _NBUF, k)

    n_steady_groups = (n_i1 - 2 * _NBUF) // _NBUF

    @pl.loop(0, n_steady_groups)
    def _grp(g):
        for s in range(_NBUF):
            k = _NBUF + g * _NBUF + s
            unit(k, s)
            wait_idx(k + _NBUF, s)
            stage_slots(s)
            start_gather(k + _NBUF, s)

    for k in range(n_i1 - _NBUF, n_i1):
        unit(k, k % _NBUF, load_ahead=False)
    for k in range(n_i1 - _NBUF, n_i1):
        wait_write(k, k % _NBUF)


def _widen_body(et_ref, o_ref):
    xt = jnp.transpose(et_ref[...], (1, 0))
    c2 = xt.shape[0] // 2
    o_ref[:, :_D] = xt[:c2, :]
    o_ref[:, _D:] = xt[c2:, :]


def _plane_body(g_ref, o_ref):
    x = g_ref[...]
    xt = jnp.transpose(x, (0, 2, 1))
    o_ref[...] = xt[:, :_D, :]


def _acc_plane_body(g_ref, acc_ref, o_ref):
    xt = jnp.transpose(g_ref[...], (0, 2, 1))
    o_ref[...] = xt[:, :_D, :]


def kernel(token_ids, embeddings):
    nv, d = embeddings.shape
    n0, n1 = token_ids.shape
    tt = token_ids.T.reshape(-1).astype(jnp.int32)

    c = _C
    table_p = pl.pallas_call(
        _widen_body,
        grid=(pl.cdiv(nv, c),),
        in_specs=[pl.BlockSpec((d, c), lambda i: (0, i))],
        out_specs=pl.BlockSpec((c // 2, 128), lambda i: (i, 0)),
        out_shape=jax.ShapeDtypeStruct(
            (pl.cdiv(nv, c) * (c // 2), 128), jnp.float32),
    )(embeddings.T)
    table_c = table_p.reshape(pl.cdiv(nv, c) * c, d)

    mesh = plsc.VectorSubcoreMesh(core_axis_name="c", subcore_axis_name="s")
    gather_rows = pl.kernel(
        functools.partial(_gather_body, n_i1=_HALF),
        out_type=jax.ShapeDtypeStruct((_HALF, n0, 128), jnp.float32),
        mesh=mesh,
        scratch_types=(
            [pltpu.VMEM((_TB,), jnp.int32) for _ in range(_NBUF)]
            + [pltpu.VMEM((_TB,), jnp.int32) for _ in range(_NBUF)]
            + [pltpu.VMEM((_TB, _D), jnp.float32) for _ in range(_NBUF)]
            + [pltpu.SemaphoreType.DMA for _ in range(3 * _NBUF)]
        ),
        compiler_params=pltpu.CompilerParams(
            use_tc_tiling_on_sc=False, needs_layout_passes=False),
    )
    piece_elems = _HALF * n0
    gs = [gather_rows(table_c, tt[j * piece_elems:(j + 1) * piece_elems])
          for j in range(_N_I1 // _HALF)]

    out_t = pl.pallas_call(
        _plane_body,
        grid=(_HALF,),
        in_specs=[pl.BlockSpec((1, n0, 128), lambda i: (i, 0, 0))],
        out_specs=pl.BlockSpec((1, _D, n0), lambda i: (i, 0, 0)),
        out_shape=jax.ShapeDtypeStruct((n1, _D, n0), jnp.float32),
    )(gs[0])
    for j in range(1, _N_I1 // _HALF):
        out_t = pl.pallas_call(
            _acc_plane_body,
            grid=(_HALF,),
            in_specs=[
                pl.BlockSpec((1, n0, 128), lambda i: (i, 0, 0)),
                pl.BlockSpec(memory_space=pltpu.HBM),
            ],
            out_specs=pl.BlockSpec(
                (1, _D, n0), lambda i, o=j * _HALF: (i + o, 0, 0)),
            out_shape=jax.ShapeDtypeStruct((n1, _D, n0), jnp.float32),
            input_output_aliases={1: 0},
        )(gs[j], out_t)
    return out_t.transpose(2, 0, 1)


# speedup vs baseline: 1.0753x; 1.0237x over previous
import functools

import jax
import jax.numpy as jnp
from jax import lax
from jax.experimental import pallas as pl
from jax.experimental.pallas import tpu as pltpu
from jax.experimental.pallas import tpu_sc as plsc

_NUM_CORES = 2
_NUM_SUBCORES = 16
_NW = _NUM_CORES * _NUM_SUBCORES
_TB = 128
_NBUF = 2

_N_I1 = 200
_C = 32768
_CLOG = 15
_N_I0 = 4096
_D = 64
_HALF = _N_I1 // 10


def _gather_body(table_hbm, tt_hbm, out_hbm, *scratch, n_i1):
    idx_b = scratch[0:_NBUF]
    p_b = scratch[_NBUF:2 * _NBUF]
    rows_b = scratch[2 * _NBUF:3 * _NBUF]
    sem_i = scratch[3 * _NBUF:4 * _NBUF]
    sem_g = scratch[4 * _NBUF:5 * _NBUF]
    sem_w = scratch[5 * _NBUF:6 * _NBUF]

    w = lax.axis_index("s") * _NUM_CORES + lax.axis_index("c")

    def start_idx(k, s):
        pltpu.async_copy(tt_hbm.at[pl.ds(k * _N_I0 + _TB * w, _TB)],
                         idx_b[s], sem_i[s])

    def wait_idx(k, s):
        pltpu.make_async_copy(tt_hbm.at[pl.ds(k * _N_I0 + _TB * w, _TB)],
                              idx_b[s], sem_i[s]).wait()

    def stage_slots(s):
        @pl.loop(0, _TB // 16)
        def _jg(jg):
            v = idx_b[s][pl.ds(jg * 16, 16)]
            slot = (lax.shift_left(lax.shift_right_logical(v, _CLOG), _CLOG)
                    + lax.shift_left(lax.bitwise_and(v, _C // 2 - 1), 1)
                    + lax.bitwise_and(lax.shift_right_logical(v, _CLOG - 1), 1))
            p_b[s][pl.ds(jg * 16, 16)] = slot

    def start_gather(k, s):
        pltpu.async_copy(table_hbm.at[p_b[s]], rows_b[s], sem_g[s])

    def wait_gather(k, s):
        pltpu.make_async_copy(table_hbm.at[p_b[s]], rows_b[s], sem_g[s]).wait()

    def start_write(k, s):
        pltpu.async_copy(
            rows_b[s], out_hbm.at[k, pl.ds(w * _TB, _TB), pl.ds(0, _D)],
            sem_w[s])

    def wait_write(k, s):
        pltpu.make_async_copy(
            rows_b[s], out_hbm.at[k, pl.ds(w * _TB, _TB), pl.ds(0, _D)],
            sem_w[s]).wait()

    def unit(k, s, with_wait_w=True, load_ahead=True):
        wait_gather(k, s)
        if with_wait_w:
            wait_write(k - _NBUF, s)
        start_write(k, s)
        if load_ahead:
            start_idx(k + _NBUF, s)

    for k in range(_NBUF):
        start_idx(k, k)
    for k in range(_NBUF):
        wait_idx(k, k)
        stage_slots(k)
        start_gather(k, k)
    for k in range(_NBUF):
        unit(k, k, with_wait_w=False)
        wait_idx(k + _NBUF, k)
        stage_slots(k)
        start_gather(k + _NBUF, k)

    n_steady_groups = (n_i1 - 2 * _NBUF) // _NBUF

    @pl.loop(0, n_steady_groups)
    def _grp(g):
        for s in range(_NBUF):
            k = _NBUF + g * _NBUF + s
            unit(k, s)
            wait_idx(k + _NBUF, s)
            stage_slots(s)
            start_gather(k + _NBUF, s)

    for k in range(n_i1 - _NBUF, n_i1):
        unit(k, k % _NBUF, load_ahead=False)
    for k in range(n_i1 - _NBUF, n_i1):
        wait_write(k, k % _NBUF)


def _widen_body(et_ref, o_ref):
    xt = jnp.transpose(et_ref[...], (1, 0))
    c2 = xt.shape[0] // 2
    o_ref[:, :_D] = xt[:c2, :]
    o_ref[:, _D:] = xt[c2:, :]


def _plane_body(g_ref, o_ref):
    x = g_ref[...]
    xt = jnp.transpose(x, (0, 2, 1))
    o_ref[...] = xt[:, :_D, :]


def _acc_plane_body(g_ref, acc_ref, o_ref):
    xt = jnp.transpose(g_ref[...], (0, 2, 1))
    o_ref[...] = xt[:, :_D, :]


def kernel(token_ids, embeddings):
    nv, d = embeddings.shape
    n0, n1 = token_ids.shape
    tt = token_ids.T.reshape(-1).astype(jnp.int32)

    c = _C
    table_p = pl.pallas_call(
        _widen_body,
        grid=(pl.cdiv(nv, c),),
        in_specs=[pl.BlockSpec((d, c), lambda i: (0, i))],
        out_specs=pl.BlockSpec((c // 2, 128), lambda i: (i, 0)),
        out_shape=jax.ShapeDtypeStruct(
            (pl.cdiv(nv, c) * (c // 2), 128), jnp.float32),
    )(embeddings.T)
    table_c = table_p.reshape(pl.cdiv(nv, c) * c, d)

    mesh = plsc.VectorSubcoreMesh(core_axis_name="c", subcore_axis_name="s")
    gather_rows = pl.kernel(
        functools.partial(_gather_body, n_i1=_HALF),
        out_type=jax.ShapeDtypeStruct((_HALF, n0, 128), jnp.float32),
        mesh=mesh,
        scratch_types=(
            [pltpu.VMEM((_TB,), jnp.int32) for _ in range(_NBUF)]
            + [pltpu.VMEM((_TB,), jnp.int32) for _ in range(_NBUF)]
            + [pltpu.VMEM((_TB, _D), jnp.float32) for _ in range(_NBUF)]
            + [pltpu.SemaphoreType.DMA for _ in range(3 * _NBUF)]
        ),
        compiler_params=pltpu.CompilerParams(
            use_tc_tiling_on_sc=False, needs_layout_passes=False),
    )
    piece_elems = _HALF * n0
    gs = [gather_rows(table_c, tt[j * piece_elems:(j + 1) * piece_elems])
          for j in range(_N_I1 // _HALF)]

    out_t = pl.pallas_call(
        _plane_body,
        grid=(_HALF,),
        in_specs=[pl.BlockSpec((1, n0, 128), lambda i: (i, 0, 0))],
        out_specs=pl.BlockSpec((1, _D, n0), lambda i: (i, 0, 0)),
        out_shape=jax.ShapeDtypeStruct((n1, _D, n0), jnp.float32),
    )(gs[0])
    for j in range(1, _N_I1 // _HALF):
        out_t = pl.pallas_call(
            _acc_plane_body,
            grid=(_HALF,),
            in_specs=[
                pl.BlockSpec((1, n0, 128), lambda i: (i, 0, 0)),
                pl.BlockSpec(memory_space=pltpu.HBM),
            ],
            out_specs=pl.BlockSpec(
                (1, _D, n0), lambda i, o=j * _HALF: (i + o, 0, 0)),
            out_shape=jax.ShapeDtypeStruct((n1, _D, n0), jnp.float32),
            input_output_aliases={1: 0},
        )(gs[j], out_t)
    return out_t.transpose(2, 0, 1)


# speedup vs baseline: 1.0784x; 1.0029x over previous
import functools

import jax
import jax.numpy as jnp
from jax import lax
from jax.experimental import pallas as pl
from jax.experimental.pallas import tpu as pltpu
from jax.experimental.pallas import tpu_sc as plsc

_NUM_CORES = 2
_NUM_SUBCORES = 16
_NW = _NUM_CORES * _NUM_SUBCORES
_TB = 128
_NBUF = 2

_N_I1 = 200
_C = 32768
_CLOG = 15
_N_I0 = 4096
_D = 64
_HALF = _N_I1 // 10


def _gather_body(table_hbm, tt_hbm, out_hbm, *scratch, n_i1):
    idx_b = scratch[0:_NBUF]
    rows_b = scratch[_NBUF:2 * _NBUF]
    sem_i = scratch[2 * _NBUF:3 * _NBUF]
    sem_g = scratch[3 * _NBUF:4 * _NBUF]
    sem_w = scratch[4 * _NBUF:5 * _NBUF]

    w = lax.axis_index("s") * _NUM_CORES + lax.axis_index("c")

    def start_idx(k, s):
        pltpu.async_copy(tt_hbm.at[pl.ds(k * _N_I0 + _TB * w, _TB)],
                         idx_b[s], sem_i[s])

    def wait_idx(k, s):
        pltpu.make_async_copy(tt_hbm.at[pl.ds(k * _N_I0 + _TB * w, _TB)],
                              idx_b[s], sem_i[s]).wait()

    def start_gather(k, s):
        pltpu.async_copy(table_hbm.at[idx_b[s]], rows_b[s], sem_g[s])

    def wait_gather(k, s):
        pltpu.make_async_copy(table_hbm.at[idx_b[s]], rows_b[s], sem_g[s]).wait()

    def start_write(k, s):
        pltpu.async_copy(
            rows_b[s], out_hbm.at[k, pl.ds(w * _TB, _TB), pl.ds(0, _D)],
            sem_w[s])

    def wait_write(k, s):
        pltpu.make_async_copy(
            rows_b[s], out_hbm.at[k, pl.ds(w * _TB, _TB), pl.ds(0, _D)],
            sem_w[s]).wait()

    def unit(k, s, load_ahead=True):
        wait_gather(k, s)
        start_write(k, s)
        if load_ahead:
            start_idx(k + _NBUF, s)

    for k in range(_NBUF):
        start_idx(k, k)
    for k in range(_NBUF):
        wait_idx(k, k)
        start_gather(k, k)
    for k in range(_NBUF):
        unit(k, k)
        wait_idx(k + _NBUF, k)
        wait_write(k, k)
        start_gather(k + _NBUF, k)

    n_steady_groups = (n_i1 - 2 * _NBUF) // _NBUF

    @pl.loop(0, n_steady_groups)
    def _grp(g):
        for s in range(_NBUF):
            k = _NBUF + g * _NBUF + s
            unit(k, s)
            wait_idx(k + _NBUF, s)
            wait_write(k, s)
            start_gather(k + _NBUF, s)

    for k in range(n_i1 - _NBUF, n_i1):
        unit(k, k % _NBUF, load_ahead=False)
    for k in range(n_i1 - _NBUF, n_i1):
        wait_write(k, k % _NBUF)


def _widen_body(et_ref, o_ref):
    xt = jnp.transpose(et_ref[...], (1, 0))
    c2 = xt.shape[0] // 2
    o_ref[:, :_D] = xt[:c2, :]
    o_ref[:, _D:] = xt[c2:, :]


def _plane_body(g_ref, o_ref):
    x = g_ref[...]
    xt = jnp.transpose(x, (0, 2, 1))
    o_ref[...] = xt[:, :_D, :]


def _acc_plane_body(g_ref, acc_ref, o_ref):
    xt = jnp.transpose(g_ref[...], (0, 2, 1))
    o_ref[...] = xt[:, :_D, :]


def kernel(token_ids, embeddings):
    nv, d = embeddings.shape
    n0, n1 = token_ids.shape
    tt = token_ids.T.reshape(-1).astype(jnp.int32)
    tt = ((tt >> _CLOG) << _CLOG) + ((tt & (_C // 2 - 1)) << 1) \
        + ((tt >> (_CLOG - 1)) & 1)

    c = _C
    table_p = pl.pallas_call(
        _widen_body,
        grid=(pl.cdiv(nv, c),),
        in_specs=[pl.BlockSpec((d, c), lambda i: (0, i))],
        out_specs=pl.BlockSpec((c // 2, 128), lambda i: (i, 0)),
        out_shape=jax.ShapeDtypeStruct(
            (pl.cdiv(nv, c) * (c // 2), 128), jnp.float32),
    )(embeddings.T)
    table_c = table_p.reshape(pl.cdiv(nv, c) * c, d)

    mesh = plsc.VectorSubcoreMesh(core_axis_name="c", subcore_axis_name="s")
    gather_rows = pl.kernel(
        functools.partial(_gather_body, n_i1=_HALF),
        out_type=jax.ShapeDtypeStruct((_HALF, n0, 128), jnp.float32),
        mesh=mesh,
        scratch_types=(
            [pltpu.VMEM((_TB,), jnp.int32) for _ in range(_NBUF)]
            + [pltpu.VMEM((_TB, _D), jnp.float32) for _ in range(_NBUF)]
            + [pltpu.SemaphoreType.DMA for _ in range(3 * _NBUF)]
        ),
        compiler_params=pltpu.CompilerParams(
            use_tc_tiling_on_sc=False, needs_layout_passes=False),
    )
    piece_elems = _HALF * n0
    gs = [gather_rows(table_c, tt[j * piece_elems:(j + 1) * piece_elems])
          for j in range(_N_I1 // _HALF)]

    out_t = pl.pallas_call(
        _plane_body,
        grid=(_HALF,),
        in_specs=[pl.BlockSpec((1, n0, 128), lambda i: (i, 0, 0))],
        out_specs=pl.BlockSpec((1, _D, n0), lambda i: (i, 0, 0)),
        out_shape=jax.ShapeDtypeStruct((n1, _D, n0), jnp.float32),
    )(gs[0])
    for j in range(1, _N_I1 // _HALF):
        out_t = pl.pallas_call(
            _acc_plane_body,
            grid=(_HALF,),
            in_specs=[
                pl.BlockSpec((1, n0, 128), lambda i: (i, 0, 0)),
                pl.BlockSpec(memory_space=pltpu.HBM),
            ],
            out_specs=pl.BlockSpec(
                (1, _D, n0), lambda i, o=j * _HALF: (i + o, 0, 0)),
            out_shape=jax.ShapeDtypeStruct((n1, _D, n0), jnp.float32),
            input_output_aliases={1: 0},
        )(gs[j], out_t)
    return out_t.transpose(2, 0, 1)
